# baseline (speedup 1.0000x reference)
import jax
import jax.numpy as jnp
from jax import lax
from jax.experimental import pallas as pl
from jax.experimental.pallas import tpu as pltpu
from jax.experimental.pallas import tpu_sc as plsc

VOCAB = 1000000
SEQ_LEN = 200
EMBED_DIM = 64
BATCH = 1024

NC, NS, L = 2, 16, 16
NW = NC * NS
IPW = BATCH // NW
IPC = 4
NCH = IPW // IPC
GSZ = 100
SPI = SEQ_LEN // GSZ
NG = IPC * SPI
SCALE = 8.0


def _body(x_hbm, table_hbm, pos8_hbm, out_hbm,
          i0, i1, g0, g1,
          sg0, sg1, so0, so1, sp0, sp1):
    wid = lax.axis_index("s") * NC + lax.axis_index("c")
    b_base = wid * IPW

    ibufs = (i0, i1)
    gbufs = (g0, g1)
    sgs = (sg0, sg1)
    sos = (so0, so1)
    sps = (sp0, sp1)

    def issue(j, sl, first=False):
        ibuf, g, sg, so, sp = ibufs[sl], gbufs[sl], sgs[sl], sos[sl], sps[sl]
        if not first:
            pltpu.make_async_copy(g, out_hbm.at[pl.ds(0, NG)], so).wait()
        for i in range(IPC):
            pltpu.async_copy(pos8_hbm, g.at[pl.ds(i * SPI, SPI)], sp)
        c0 = (b_base + j * IPC) * SPI
        pltpu.sync_copy(x_hbm.at[pl.ds(c0, NG)], ibuf)
        for i in range(IPC):
            pltpu.make_async_copy(
                pos8_hbm, g.at[pl.ds(i * SPI, SPI)], sp).wait()
        for k in range(NG):
            pltpu.async_copy(
                table_hbm.at[ibuf.at[k]], g.at[k], sg, add=True)

    def consume(j, sl):
        ibuf, g, sg, so = ibufs[sl], gbufs[sl], sgs[sl], sos[sl]
        for k in range(NG):
            pltpu.make_async_copy(
                table_hbm.at[ibuf.at[k]], g.at[k], sg).wait()
        for k in range(NG):
            def row_body(r, carry, _k=k):
                for q in range(EMBED_DIM // L):
                    qs = pl.ds(q * L, L)
                    g[_k, r, qs] = g[_k, r, qs] * SCALE
                return carry
            lax.fori_loop(0, GSZ, row_body, 0, unroll=10)
        c0 = (b_base + j * IPC) * SPI
        pltpu.async_copy(g, out_hbm.at[pl.ds(c0, NG)], so)

    issue(0, 0, first=True)

    def pair_body(k, carry):
        @pl.when(k == 0)
        def _():
            issue(1, 1, first=True)

        @pl.when(k > 0)
        def _():
            issue(2 * k + 1, 1)

        consume(2 * k, 0)

        @pl.when(k < NCH // 2 - 1)
        def _():
            issue(2 * k + 2, 0)

        consume(2 * k + 1, 1)
        return carry

    lax.fori_loop(0, NCH // 2, pair_body, 0)

    for sl in range(2):
        pltpu.make_async_copy(
            gbufs[sl], out_hbm.at[pl.ds(0, NG)], sos[sl]).wait()


@jax.jit
def kernel(x, token_table, pos_table):
    x2 = x.reshape(BATCH * SEQ_LEN // GSZ, GSZ)
    pos8 = (pos_table * (1.0 / SCALE)).reshape(SPI, GSZ, EMBED_DIM)
    mesh = plsc.VectorSubcoreMesh(
        core_axis_name="c", subcore_axis_name="s",
        num_cores=NC, num_subcores=NS)
    out = pl.kernel(
        _body,
        out_type=jax.ShapeDtypeStruct(
            (BATCH * SEQ_LEN // GSZ, GSZ, EMBED_DIM), jnp.float32),
        mesh=mesh,
        scratch_types=[
            pltpu.VMEM((NG, GSZ), jnp.int32),
            pltpu.VMEM((NG, GSZ), jnp.int32),
            pltpu.VMEM((NG, GSZ, EMBED_DIM), jnp.float32),
            pltpu.VMEM((NG, GSZ, EMBED_DIM), jnp.float32),
            pltpu.SemaphoreType.DMA,
            pltpu.SemaphoreType.DMA,
            pltpu.SemaphoreType.DMA,
            pltpu.SemaphoreType.DMA,
            pltpu.SemaphoreType.DMA,
            pltpu.SemaphoreType.DMA,
        ],
        compiler_params=pltpu.CompilerParams(use_tc_tiling_on_sc=False),
    )(x2, token_table, pos8)
    return out.reshape(BATCH, SEQ_LEN, EMBED_DIM)


# speedup vs baseline: 1.0384x; 1.0384x over previous
import jax
import jax.numpy as jnp
from jax import lax
from jax.experimental import pallas as pl
from jax.experimental.pallas import tpu as pltpu
from jax.experimental.pallas import tpu_sc as plsc

VOCAB = 1000000
SEQ_LEN = 200
EMBED_DIM = 64
BATCH = 1024

NC, NS, L = 2, 16, 16
NW = NC * NS
IPW = BATCH // NW
IPC = 2
NCH = IPW // IPC
GSZ = 100
SPI = SEQ_LEN // GSZ
NG = IPC * SPI
NSL = 3
SCALE = 8.0


def _body(x_hbm, table_hbm, pos8_hbm, out_hbm, *refs):
    ibufs = refs[0:NSL]
    gbufs = refs[NSL:2 * NSL]
    sps = refs[2 * NSL:3 * NSL]
    sis = refs[3 * NSL:4 * NSL]
    sgs = refs[4 * NSL:5 * NSL]
    sos = refs[5 * NSL:6 * NSL]

    wid = lax.axis_index("s") * NC + lax.axis_index("c")
    c_base = wid * IPW * SPI

    def pre(j):
        sl = j % NSL
        ibuf, g = ibufs[sl], gbufs[sl]
        if j >= NSL:
            pltpu.make_async_copy(g, out_hbm.at[pl.ds(0, NG)],
                                  sos[sl]).wait()
        pltpu.async_copy(pos8_hbm, g, sps[sl])
        pltpu.async_copy(x_hbm.at[pl.ds(c_base + j * NG, NG)], ibuf,
                         sis[sl])

    def gather(j):
        sl = j % NSL
        ibuf, g = ibufs[sl], gbufs[sl]
        pltpu.make_async_copy(pos8_hbm, g, sps[sl]).wait()
        pltpu.make_async_copy(x_hbm.at[pl.ds(0, NG)], ibuf, sis[sl]).wait()
        for k in range(NG):
            pltpu.async_copy(
                table_hbm.at[ibuf.at[k]], g.at[k], sgs[sl], add=True)

    def consume(j):
        sl = j % NSL
        ibuf, g = ibufs[sl], gbufs[sl]
        for k in range(NG):
            pltpu.make_async_copy(
                table_hbm.at[ibuf.at[k]], g.at[k], sgs[sl]).wait()

        def row_body(r, carry):
            for k in range(NG):
                for q in range(EMBED_DIM // L):
                    qs = pl.ds(q * L, L)
                    g[k, r, qs] = g[k, r, qs] * SCALE
            return carry

        lax.fori_loop(0, GSZ, row_body, 0, unroll=2)
        pltpu.async_copy(g, out_hbm.at[pl.ds(c_base + j * NG, NG)],
                         sos[sl])

    for j in range(NCH + 2):
        if j < NCH:
            pre(j)
        if 1 <= j <= NCH:
            gather(j - 1)
        if j >= 2:
            consume(j - 2)

    for j in range(NCH - NSL, NCH):
        sl = j % NSL
        pltpu.make_async_copy(gbufs[sl], out_hbm.at[pl.ds(0, NG)],
                              sos[sl]).wait()


@jax.jit
def kernel(x, token_table, pos_table):
    x2 = x.reshape(BATCH * SEQ_LEN // GSZ, GSZ)
    pos8 = (pos_table * (1.0 / SCALE)).reshape(SPI, GSZ, EMBED_DIM)
    pos8t = jnp.tile(pos8, (IPC, 1, 1))
    mesh = plsc.VectorSubcoreMesh(
        core_axis_name="c", subcore_axis_name="s",
        num_cores=NC, num_subcores=NS)
    scratch = (
        [pltpu.VMEM((NG, GSZ), jnp.int32)] * NSL
        + [pltpu.VMEM((NG, GSZ, EMBED_DIM), jnp.float32)] * NSL
        + [pltpu.SemaphoreType.DMA] * (4 * NSL)
    )
    out = pl.kernel(
        _body,
        out_type=jax.ShapeDtypeStruct(
            (BATCH * SEQ_LEN // GSZ, GSZ, EMBED_DIM), jnp.float32),
        mesh=mesh,
        scratch_types=scratch,
        compiler_params=pltpu.CompilerParams(use_tc_tiling_on_sc=False),
    )(x2, token_table, pos8t)
    return out.reshape(BATCH, SEQ_LEN, EMBED_DIM)
